# Initial kernel scaffold; baseline (speedup 1.0000x reference)
import jax
import jax.numpy as jnp
from jax.experimental import pallas as pl


def kernel(scores):
    raise NotImplementedError("write your pallas kernel here")


# speedup vs baseline: 5.9307x; 5.9307x over previous
import math

import jax
import jax.numpy as jnp
from jax.experimental import pallas as pl
from jax.experimental.pallas import tpu as pltpu

_K = 21
_B = 4096
_BLK = 128
_NBLK = _B // _BLK


def _tc_body(x_ref, out_ref, acc_ref):
    pid = pl.program_id(0)

    @pl.when(pid == 0)
    def _():
        acc_ref[0] = 0.0
        acc_ref[1] = 0.0

    x = x_ref[...]
    rows = jax.lax.broadcasted_iota(jnp.int32, (_BLK, _B), 0) + pid * _BLK
    cols = jax.lax.broadcasted_iota(jnp.int32, (_BLK, _B), 1)
    diag = cols == rows
    pos = jnp.sum(jnp.where(diag, x, 0.0), axis=1, keepdims=True)
    labels = rows[:, :1]

    neg = jnp.float32(-jnp.inf)
    bigi = jnp.int32(2**30)
    lv = jnp.full((_BLK, 1), jnp.inf, jnp.float32)
    li = jnp.full((_BLK, 1), -1, jnp.int32)
    loss = jnp.float32(0.0)
    msum = jnp.float32(0.0)
    for r in range(_K):
        valid = (x < lv) | ((x == lv) & (cols > li))
        m = jnp.max(jnp.where(valid, x, neg), axis=1, keepdims=True)
        idx = jnp.min(jnp.where(valid & (x == m), cols, bigi), axis=1,
                      keepdims=True)
        w = jnp.float32(1.0 / math.log2(r + 2.0))
        notself = idx != labels
        d = m - pos
        sp = jnp.maximum(d, 0.0) + jnp.log1p(jnp.exp(-jnp.abs(d)))
        loss = loss + jnp.sum(jnp.where(notself, sp, 0.0)) * w
        msum = msum + jnp.sum(notself.astype(jnp.float32))
        lv, li = m, idx
    acc_ref[0] += loss
    acc_ref[1] += msum

    @pl.when(pid == _NBLK - 1)
    def _():
        out_ref[0, 0] = acc_ref[0] / jnp.maximum(acc_ref[1], 1.0)


@jax.jit
def kernel(scores):
    out = pl.pallas_call(
        _tc_body,
        grid=(_NBLK,),
        in_specs=[pl.BlockSpec((_BLK, _B), lambda i: (i, 0))],
        out_specs=pl.BlockSpec(memory_space=pltpu.SMEM),
        out_shape=jax.ShapeDtypeStruct((1, 1), jnp.float32),
        scratch_shapes=[pltpu.SMEM((2,), jnp.float32)],
    )(scores)
    return out[0, 0]


# speedup vs baseline: 9.5205x; 1.6053x over previous
import functools
import math

import jax
import jax.numpy as jnp
from jax import lax
from jax.experimental import pallas as pl
from jax.experimental.pallas import tpu as pltpu
from jax.experimental.pallas import tpu_sc as plsc

_B = 4096
_K = 21
_KO = 32
_L = 16
_NC = 2
_NS = 16
_NW = _NC * _NS
_RPW = _B // _NW
_NCHUNK = _B // _L
_CTHRESH = 2.25


def _sc_body(scores_hbm, vals_hbm, inds_hbm,
             row_buf0, row_buf1, pool_v, pool_i, out_v, out_i, sem0, sem1):
    wid = lax.axis_index("s") * _NC + lax.axis_index("c")
    row0 = wid * _RPW
    iota = lax.broadcasted_iota(jnp.int32, (_L,), 0)
    ninf = jnp.float32(-jnp.inf)
    bigi = jnp.int32(2**30)
    ninf_v = jnp.full((_L,), ninf, jnp.float32)
    bigi_v = jnp.full((_L,), bigi, jnp.int32)

    def process(row, r):
        glabel = row0 + r

        def stat_body(j, carry):
            s, s2 = carry
            v = row[pl.ds(j * (_L * _L), _L)]
            return s + v, s2 + v * v

        zs = jnp.zeros((_L,), jnp.float32)
        s, s2 = lax.fori_loop(0, _L, stat_body, (zs, zs))
        rns = jnp.float32(1.0 / (_L * _L))
        mean = plsc.cumsum(s)[_L - 1] * rns
        var = jnp.maximum(plsc.cumsum(s2)[_L - 1] * rns - mean * mean, 0.0)
        ib = lax.bitcast_convert_type(var, jnp.int32)
        y = lax.bitcast_convert_type(jnp.int32(0x5F3759DF) - (ib >> 1),
                                     jnp.float32)
        for _ in range(3):
            y = y * (1.5 - 0.5 * var * y * y)
        sigma = var * y
        thr = mean + _CTHRESH * sigma

        def collect(t):
            tv = jnp.full((_L,), t, jnp.float32)

            def body(j, off):
                v = row[pl.ds(j * _L, _L)]
                m = v > tv
                iv = iota + j * _L
                plsc.store_compressed(pool_v.at[pl.ds(off, _L)], v, mask=m)
                plsc.store_compressed(pool_i.at[pl.ds(off, _L)], iv, mask=m)
                return off + plsc.all_reduce_population_count(m)[0]

            return lax.fori_loop(0, _NCHUNK, body, jnp.int32(0))

        n = collect(thr)
        n = lax.cond(n < _K, lambda: collect(ninf), lambda: n)

        pool_v[pl.ds(n, _L)] = ninf_v
        pool_i[pl.ds(n, _L)] = bigi_v

        def merge_body(c, st):
            av, ai, bv, bi = st
            v = pool_v[pl.ds(c * _L, _L)]
            iv = pool_i[pl.ds(c * _L, _L)]
            vs, ivs = plsc.sort_key_val(v, iv, descending=False)
            sel1 = (bv > vs) | ((bv == vs) & (bi < ivs))
            lv = jnp.where(sel1, bv, vs)
            li = jnp.where(sel1, bi, ivs)
            lvs, lis = plsc.sort_key_val(lv, li, descending=False)
            sel2 = (av > lvs) | ((av == lvs) & (ai < lis))
            hv = jnp.where(sel2, av, lvs)
            hi = jnp.where(sel2, ai, lis)
            lov = jnp.where(sel2, lvs, av)
            loi = jnp.where(sel2, lis, ai)
            av, ai = plsc.sort_key_val(hv, hi, descending=True)
            bv, bi = plsc.sort_key_val(lov, loi, descending=True)
            return av, ai, bv, bi

        nchunks = (n + _L - 1) >> 4
        av, ai, bv, bi = lax.fori_loop(
            0, nchunks, merge_body, (ninf_v, bigi_v, ninf_v, bigi_v))

        chunk = row[pl.ds((glabel >> 4) * _L, _L)]
        lane_m = (iota == (glabel & (_L - 1))).astype(jnp.int32)
        cb = lax.bitcast_convert_type(chunk, jnp.int32) & (0 - lane_m)
        pos = lax.bitcast_convert_type(plsc.cumsum(cb)[_L - 1], jnp.float32)
        bv = jnp.where(iota == _L - 1, jnp.full((_L,), pos, jnp.float32), bv)
        out_v[r, pl.ds(0, _L)] = av
        out_v[r, pl.ds(_L, _L)] = bv
        out_i[r, pl.ds(0, _L)] = ai
        out_i[r, pl.ds(_L, _L)] = bi

    pltpu.async_copy(scores_hbm.at[row0], row_buf0, sem0)
    pltpu.async_copy(scores_hbm.at[row0 + 1], row_buf1, sem1)

    def pair_body(g, _):
        for b, (row, sem) in enumerate(((row_buf0, sem0), (row_buf1, sem1))):
            r = g * 2 + b
            pltpu.make_async_copy(scores_hbm.at[row0 + r], row, sem).wait()
            process(row, r)

            @pl.when(g < _RPW // 2 - 1)
            def _():
                pltpu.async_copy(scores_hbm.at[row0 + r + 2], row, sem)
        return 0

    lax.fori_loop(0, _RPW // 2, pair_body, 0)

    pltpu.sync_copy(out_v, vals_hbm.at[pl.ds(row0, _RPW)])
    pltpu.sync_copy(out_i, inds_hbm.at[pl.ds(row0, _RPW)])


def _loss_body(v_ref, i_ref, out_ref):
    v = v_ref[...]
    idx = i_ref[...]
    pos = v[:, _KO - 1:_KO]
    rows = lax.broadcasted_iota(jnp.int32, (_B, _KO), 0)
    cols = lax.broadcasted_iota(jnp.int32, (_B, _KO), 1)
    w = jnp.float32(math.log(2.0)) / jnp.log(cols.astype(jnp.float32) + 2.0)
    valid = (cols < _K) & (idx != rows)
    d = v - pos
    sp = jnp.maximum(d, 0.0) + jnp.log1p(jnp.exp(-jnp.abs(d)))
    num = jnp.sum(jnp.where(valid, sp * w, 0.0))
    den = jnp.sum(valid.astype(jnp.float32))
    out_ref[0, 0] = num / jnp.maximum(den, 1.0)


@jax.jit
def kernel(scores):
    mesh = plsc.VectorSubcoreMesh(core_axis_name="c", subcore_axis_name="s",
                                  num_cores=_NC, num_subcores=_NS)
    sc_call = pl.kernel(
        _sc_body,
        out_type=[
            jax.ShapeDtypeStruct((_B, _KO), jnp.float32),
            jax.ShapeDtypeStruct((_B, _KO), jnp.int32),
        ],
        mesh=mesh,
        compiler_params=pltpu.CompilerParams(needs_layout_passes=False),
        scratch_types=[
            pltpu.VMEM((_B,), jnp.float32),
            pltpu.VMEM((_B,), jnp.float32),
            pltpu.VMEM((_B + _L,), jnp.float32),
            pltpu.VMEM((_B + _L,), jnp.int32),
            pltpu.VMEM((_RPW, _KO), jnp.float32),
            pltpu.VMEM((_RPW, _KO), jnp.int32),
            pltpu.SemaphoreType.DMA,
            pltpu.SemaphoreType.DMA,
        ],
    )
    vals, inds = sc_call(scores)
    loss = pl.pallas_call(
        _loss_body,
        out_specs=pl.BlockSpec(memory_space=pltpu.SMEM),
        out_shape=jax.ShapeDtypeStruct((1, 1), jnp.float32),
    )(vals, inds)
    return loss[0, 0]


# speedup vs baseline: 10.8052x; 1.1349x over previous
import functools
import math

import jax
import jax.numpy as jnp
from jax import lax
from jax.experimental import pallas as pl
from jax.experimental.pallas import tpu as pltpu
from jax.experimental.pallas import tpu_sc as plsc

_B = 4096
_K = 21
_KO = 32
_L = 16
_NC = 2
_NS = 16
_NW = _NC * _NS
_RPW = _B // _NW
_NCHUNK = _B // _L
_CTHRESH = 2.25


def _sc_body(scores_hbm, vals_hbm, inds_hbm,
             row_buf0, row_buf1, pool_v, pool_i, out_v, out_i, sem0, sem1):
    wid = lax.axis_index("s") * _NC + lax.axis_index("c")
    row0 = wid * _RPW
    iota = lax.broadcasted_iota(jnp.int32, (_L,), 0)
    ninf = jnp.float32(-jnp.inf)
    bigi = jnp.int32(2**30)
    ninf_v = jnp.full((_L,), ninf, jnp.float32)
    bigi_v = jnp.full((_L,), bigi, jnp.int32)

    def process(row, r):
        glabel = row0 + r

        def stat_body(j, carry):
            s, s2 = carry
            v = row[pl.ds(j * (_L * _L), _L)]
            return s + v, s2 + v * v

        zs = jnp.zeros((_L,), jnp.float32)
        s, s2 = lax.fori_loop(0, _L, stat_body, (zs, zs), unroll=4)
        rns = jnp.float32(1.0 / (_L * _L))
        mean = plsc.cumsum(s)[_L - 1] * rns
        var = jnp.maximum(plsc.cumsum(s2)[_L - 1] * rns - mean * mean, 0.0)
        ib = lax.bitcast_convert_type(var, jnp.int32)
        y = lax.bitcast_convert_type(jnp.int32(0x5F3759DF) - (ib >> 1),
                                     jnp.float32)
        for _ in range(3):
            y = y * (1.5 - 0.5 * var * y * y)
        sigma = var * y
        thr = mean + _CTHRESH * sigma

        def collect(t):
            tv = jnp.full((_L,), t, jnp.float32)

            def body(j, off):
                v = row[pl.ds(j * _L, _L)]
                m = v > tv
                iv = iota + j * _L
                plsc.store_compressed(pool_v.at[pl.ds(off, _L)], v, mask=m)
                plsc.store_compressed(pool_i.at[pl.ds(off, _L)], iv, mask=m)
                return off + plsc.all_reduce_population_count(m)[0]

            return lax.fori_loop(0, _NCHUNK, body, jnp.int32(0), unroll=8)

        n = collect(thr)
        n = lax.cond(n < _K, lambda: collect(ninf), lambda: n)

        pool_v[pl.ds(n, _L)] = ninf_v
        pool_i[pl.ds(n, _L)] = bigi_v

        def merge_body(c, st):
            av, ai, bv, bi = st
            v = pool_v[pl.ds(c * _L, _L)]
            iv = pool_i[pl.ds(c * _L, _L)]
            vs, ivs = plsc.sort_key_val(v, iv, descending=False)
            sel1 = (bv > vs) | ((bv == vs) & (bi < ivs))
            lv = jnp.where(sel1, bv, vs)
            li = jnp.where(sel1, bi, ivs)
            lvs, lis = plsc.sort_key_val(lv, li, descending=False)
            sel2 = (av > lvs) | ((av == lvs) & (ai < lis))
            hv = jnp.where(sel2, av, lvs)
            hi = jnp.where(sel2, ai, lis)
            lov = jnp.where(sel2, lvs, av)
            loi = jnp.where(sel2, lis, ai)
            av, ai = plsc.sort_key_val(hv, hi, descending=True)
            bv, bi = plsc.sort_key_val(lov, loi, descending=True)
            return av, ai, bv, bi

        nchunks = (n + _L - 1) >> 4
        av, ai, bv, bi = lax.fori_loop(
            0, nchunks, merge_body, (ninf_v, bigi_v, ninf_v, bigi_v))

        chunk = row[pl.ds((glabel >> 4) * _L, _L)]
        lane_m = (iota == (glabel & (_L - 1))).astype(jnp.int32)
        cb = lax.bitcast_convert_type(chunk, jnp.int32) & (0 - lane_m)
        pos = lax.bitcast_convert_type(plsc.cumsum(cb)[_L - 1], jnp.float32)
        bv = jnp.where(iota == _L - 1, jnp.full((_L,), pos, jnp.float32), bv)
        out_v[r, pl.ds(0, _L)] = av
        out_v[r, pl.ds(_L, _L)] = bv
        out_i[r, pl.ds(0, _L)] = ai
        out_i[r, pl.ds(_L, _L)] = bi

    pltpu.async_copy(scores_hbm.at[row0], row_buf0, sem0)
    pltpu.async_copy(scores_hbm.at[row0 + 1], row_buf1, sem1)

    def pair_body(g, _):
        for b, (row, sem) in enumerate(((row_buf0, sem0), (row_buf1, sem1))):
            r = g * 2 + b
            pltpu.make_async_copy(scores_hbm.at[row0 + r], row, sem).wait()
            process(row, r)

            @pl.when(g < _RPW // 2 - 1)
            def _():
                pltpu.async_copy(scores_hbm.at[row0 + r + 2], row, sem)
        return 0

    lax.fori_loop(0, _RPW // 2, pair_body, 0)

    pltpu.sync_copy(out_v, vals_hbm.at[pl.ds(row0, _RPW)])
    pltpu.sync_copy(out_i, inds_hbm.at[pl.ds(row0, _RPW)])


def _loss_body(v_ref, i_ref, out_ref):
    v = v_ref[...]
    idx = i_ref[...]
    pos = v[:, _KO - 1:_KO]
    rows = lax.broadcasted_iota(jnp.int32, (_B, _KO), 0)
    cols = lax.broadcasted_iota(jnp.int32, (_B, _KO), 1)
    w = jnp.float32(math.log(2.0)) / jnp.log(cols.astype(jnp.float32) + 2.0)
    valid = (cols < _K) & (idx != rows)
    d = v - pos
    sp = jnp.maximum(d, 0.0) + jnp.log1p(jnp.exp(-jnp.abs(d)))
    num = jnp.sum(jnp.where(valid, sp * w, 0.0))
    den = jnp.sum(valid.astype(jnp.float32))
    out_ref[0, 0] = num / jnp.maximum(den, 1.0)


@jax.jit
def kernel(scores):
    mesh = plsc.VectorSubcoreMesh(core_axis_name="c", subcore_axis_name="s",
                                  num_cores=_NC, num_subcores=_NS)
    sc_call = pl.kernel(
        _sc_body,
        out_type=[
            jax.ShapeDtypeStruct((_B, _KO), jnp.float32),
            jax.ShapeDtypeStruct((_B, _KO), jnp.int32),
        ],
        mesh=mesh,
        compiler_params=pltpu.CompilerParams(needs_layout_passes=False),
        scratch_types=[
            pltpu.VMEM((_B,), jnp.float32),
            pltpu.VMEM((_B,), jnp.float32),
            pltpu.VMEM((_B + _L,), jnp.float32),
            pltpu.VMEM((_B + _L,), jnp.int32),
            pltpu.VMEM((_RPW, _KO), jnp.float32),
            pltpu.VMEM((_RPW, _KO), jnp.int32),
            pltpu.SemaphoreType.DMA,
            pltpu.SemaphoreType.DMA,
        ],
    )
    vals, inds = sc_call(scores)
    loss = pl.pallas_call(
        _loss_body,
        out_specs=pl.BlockSpec(memory_space=pltpu.SMEM),
        out_shape=jax.ShapeDtypeStruct((1, 1), jnp.float32),
    )(vals, inds)
    return loss[0, 0]
